# baseline (speedup 1.0000x reference)
import functools

import jax
import jax.numpy as jnp
from jax import lax
from jax.experimental import pallas as pl
from jax.experimental.pallas import tpu as pltpu
from jax.experimental.pallas import tpu_sc as plsc

N = 10000
E = 320000
D = 128
NC, NS = 2, 16
EPT = E // (NC * NS)
K = 125
NCHUNK = EPT // K
CPG = 16
G = NCHUNK // CPG
AK = 25
ANCHUNK = EPT // AK
ACPG = 40
AG = ANCHUNK // ACPG
NBUF = 8
NACC = 10240
RPT = NACC // NS
NDEG = 10240
DPT = NDEG // NS
BM = 400
GRID = N // BM

_mesh = plsc.VectorSubcoreMesh(core_axis_name="c", subcore_axis_name="s")


@functools.partial(
    pl.kernel,
    mesh=_mesh,
    out_type=jax.ShapeDtypeStruct((NC * NDEG,), jnp.float32),
    scratch_types=[
        pltpu.VMEM((NCHUNK, K), jnp.int32),
        pltpu.VMEM((128,), jnp.float32),
        pltpu.VMEM((DPT,), jnp.float32),
        pltpu.VMEM_SHARED((NDEG,), jnp.float32),
    ],
)
def _deg_kernel(dst_hbm, out_hbm, dst_v, ones_v, zbuf, acc):
    c = lax.axis_index("c")
    s = lax.axis_index("s")
    pltpu.sync_copy(dst_hbm.at[c, s], dst_v)

    @pl.loop(0, 8)
    def _fill_ones(i):
        ones_v[pl.ds(i * 16, 16)] = jnp.ones((16,), jnp.float32)

    @pl.loop(0, DPT // 16)
    def _fill_zero(i):
        zbuf[pl.ds(i * 16, 16)] = jnp.zeros((16,), jnp.float32)

    pltpu.sync_copy(zbuf, acc.at[pl.ds(s * DPT, DPT)])
    plsc.subcore_barrier()

    @pl.loop(0, NCHUNK)
    def _scatter(j):
        pltpu.sync_copy(ones_v.at[pl.ds(0, K)], acc.at[dst_v.at[j]], add=True)

    plsc.subcore_barrier()
    pltpu.sync_copy(acc.at[pl.ds(s * DPT, DPT)],
                    out_hbm.at[pl.ds(c * NDEG + s * DPT, DPT)])


@functools.partial(
    pl.kernel,
    mesh=_mesh,
    out_type=jax.ShapeDtypeStruct((NC, NACC, D), jnp.float32),
    scratch_types=(
        [pltpu.VMEM((ACPG, AK), jnp.int32)] * 2
        + [pltpu.VMEM((AK, D), jnp.float32)] * NBUF
        + [pltpu.VMEM((8, D), jnp.float32)]
        + [pltpu.VMEM_SHARED((NACC, D), jnp.float32)]
        + [pltpu.SemaphoreType.DMA] * NBUF
    ),
)
def _agg_kernel(g_hbm, src_hbm, dst_hbm, out_hbm, *refs):
    src_v, dst_v = refs[0], refs[1]
    rows = refs[2:2 + NBUF]
    zbuf = refs[2 + NBUF]
    acc = refs[3 + NBUF]
    sems = refs[4 + NBUF:]
    c = lax.axis_index("c")
    s = lax.axis_index("s")

    @pl.loop(0, 8)
    def _fill_zero(i):
        @pl.loop(0, D // 16)
        def _inner(k):
            zbuf[i, pl.ds(k * 16, 16)] = jnp.zeros((16,), jnp.float32)

    @pl.loop(0, RPT // 8)
    def _zero_acc(i):
        pltpu.sync_copy(zbuf, acc.at[pl.ds(s * RPT + i * 8, 8)])

    plsc.subcore_barrier()

    @pl.loop(0, AG)
    def _grp(g):
        pltpu.sync_copy(src_hbm.at[c, s, g], src_v)
        pltpu.sync_copy(dst_hbm.at[c, s, g], dst_v)
        for b in range(NBUF - 1):
            pltpu.async_copy(g_hbm.at[src_v.at[b]], rows[b], sems[b])

        @pl.loop(0, ACPG, step=NBUF)
        def _edges(j):
            for b in range(NBUF):
                pltpu.make_async_copy(
                    g_hbm.at[src_v.at[j + b]], rows[b], sems[b]).wait()

                @pl.when(j + b + NBUF - 1 < ACPG)
                def _issue():
                    nb = (b + NBUF - 1) % NBUF
                    pltpu.async_copy(
                        g_hbm.at[src_v.at[j + b + NBUF - 1]], rows[nb], sems[nb])

                pltpu.sync_copy(rows[b], acc.at[dst_v.at[j + b]], add=True)

    plsc.subcore_barrier()
    pltpu.sync_copy(acc.at[pl.ds(s * RPT, RPT)], out_hbm.at[c, pl.ds(s * RPT, RPT)])


def _tc_first_body(x_ref, w_ref, p0_ref, p1_ref, g_ref, dinv_ref):
    d = lax.rsqrt(p0_ref[...] + p1_ref[...] + 1.0)
    xw = jnp.dot(x_ref[...], w_ref[...], preferred_element_type=jnp.float32)
    g_ref[...] = xw * d
    dinv_ref[...] = d


def _tc_mid_body(p0_ref, p1_ref, g_ref, dinv_ref, b_ref, w_ref, gout_ref):
    d = dinv_ref[...]
    y = d * (p0_ref[...] + p1_ref[...] + g_ref[...]) + b_ref[...]
    h = jnp.maximum(y, 0.0)
    gout_ref[...] = jnp.dot(h, w_ref[...], preferred_element_type=jnp.float32) * d


def _tc_last_body(p0_ref, p1_ref, g_ref, dinv_ref, b_ref, out_ref):
    d = dinv_ref[...]
    out_ref[...] = d * (p0_ref[...] + p1_ref[...] + g_ref[...]) + b_ref[...]


def _rows(i):
    return (i, 0)


def _same(i):
    return (0, 0)


_b_rows = pl.BlockSpec((BM, D), _rows)
_b_col = pl.BlockSpec((BM, 1), _rows)
_b_w = pl.BlockSpec((D, D), _same)
_b_bias = pl.BlockSpec((1, D), _same)

_tc_first = pl.pallas_call(
    _tc_first_body,
    grid=(GRID,),
    in_specs=[_b_rows, _b_w, _b_col, _b_col],
    out_specs=[_b_rows, _b_col],
    out_shape=[
        jax.ShapeDtypeStruct((N, D), jnp.float32),
        jax.ShapeDtypeStruct((N, 1), jnp.float32),
    ],
)

_tc_mid = pl.pallas_call(
    _tc_mid_body,
    grid=(GRID,),
    in_specs=[_b_rows, _b_rows, _b_rows, _b_col, _b_bias, _b_w],
    out_specs=_b_rows,
    out_shape=jax.ShapeDtypeStruct((N, D), jnp.float32),
)

_tc_last = pl.pallas_call(
    _tc_last_body,
    grid=(GRID,),
    in_specs=[_b_rows, _b_rows, _b_rows, _b_col, _b_bias],
    out_specs=_b_rows,
    out_shape=jax.ShapeDtypeStruct((N, D), jnp.float32),
)


@jax.jit
def kernel(x, edge_index, W1, b1, W2, b2, W3, b3):
    ei = edge_index.astype(jnp.int32)
    src = ei[0].reshape(NC, NS, NCHUNK, K)
    dst = ei[1].reshape(NC, NS, NCHUNK, K)
    src_a = ei[0].reshape(NC, NS, AG, ACPG, AK)
    dst_a = ei[1].reshape(NC, NS, AG, ACPG, AK)

    deg = _deg_kernel(dst)
    p0 = deg[:N].reshape(N, 1)
    p1 = deg[NDEG:NDEG + N].reshape(N, 1)

    g1, dinv = _tc_first(x, W1, p0, p1)
    a1 = _agg_kernel(g1, src_a, dst_a)
    g2 = _tc_mid(a1[0], a1[1], g1, dinv, b1.reshape(1, D), W2)
    a2 = _agg_kernel(g2, src_a, dst_a)
    g3 = _tc_mid(a2[0], a2[1], g2, dinv, b2.reshape(1, D), W3)
    a3 = _agg_kernel(g3, src_a, dst_a)
    out = _tc_last(a3[0], a3[1], g3, dinv, b3.reshape(1, D))
    return out


# speedup vs baseline: 1.1320x; 1.1320x over previous
import functools

import jax
import jax.numpy as jnp
from jax import lax
from jax.experimental import pallas as pl
from jax.experimental.pallas import tpu as pltpu
from jax.experimental.pallas import tpu_sc as plsc

N = 10000
E = 320000
D = 128
NC, NS = 2, 16
EPT = E // (NC * NS)
K = 125
NCHUNK = EPT // K
CPG = 16
G = NCHUNK // CPG
AK = 50
ANCHUNK = EPT // AK
ACPG = 40
AG = ANCHUNK // ACPG
NBUF = 5
NACC = 10240
RPT = NACC // NS
NDEG = 10240
DPT = NDEG // NS
BM = 400
GRID = N // BM

_mesh = plsc.VectorSubcoreMesh(core_axis_name="c", subcore_axis_name="s")


@functools.partial(
    pl.kernel,
    mesh=_mesh,
    out_type=jax.ShapeDtypeStruct((NC * NDEG,), jnp.float32),
    scratch_types=[
        pltpu.VMEM((NCHUNK, K), jnp.int32),
        pltpu.VMEM((128,), jnp.float32),
        pltpu.VMEM((DPT,), jnp.float32),
        pltpu.VMEM_SHARED((NDEG,), jnp.float32),
    ],
)
def _deg_kernel(dst_hbm, out_hbm, dst_v, ones_v, zbuf, acc):
    c = lax.axis_index("c")
    s = lax.axis_index("s")
    pltpu.sync_copy(dst_hbm.at[c, s], dst_v)

    @pl.loop(0, 8)
    def _fill_ones(i):
        ones_v[pl.ds(i * 16, 16)] = jnp.ones((16,), jnp.float32)

    @pl.loop(0, DPT // 16)
    def _fill_zero(i):
        zbuf[pl.ds(i * 16, 16)] = jnp.zeros((16,), jnp.float32)

    pltpu.sync_copy(zbuf, acc.at[pl.ds(s * DPT, DPT)])
    plsc.subcore_barrier()

    @pl.loop(0, NCHUNK)
    def _scatter(j):
        pltpu.sync_copy(ones_v.at[pl.ds(0, K)], acc.at[dst_v.at[j]], add=True)

    plsc.subcore_barrier()
    pltpu.sync_copy(acc.at[pl.ds(s * DPT, DPT)],
                    out_hbm.at[pl.ds(c * NDEG + s * DPT, DPT)])


@functools.partial(
    pl.kernel,
    mesh=_mesh,
    out_type=jax.ShapeDtypeStruct((NC, NACC, D), jnp.float32),
    scratch_types=(
        [pltpu.VMEM((ACPG, AK), jnp.int32)] * 2
        + [pltpu.VMEM((AK, D), jnp.float32)] * NBUF
        + [pltpu.VMEM((8, D), jnp.float32)]
        + [pltpu.VMEM_SHARED((NACC, D), jnp.float32)]
        + [pltpu.SemaphoreType.DMA] * NBUF
    ),
)
def _agg_kernel(g_hbm, src_hbm, dst_hbm, out_hbm, *refs):
    src_v, dst_v = refs[0], refs[1]
    rows = refs[2:2 + NBUF]
    zbuf = refs[2 + NBUF]
    acc = refs[3 + NBUF]
    sems = refs[4 + NBUF:]
    c = lax.axis_index("c")
    s = lax.axis_index("s")

    @pl.loop(0, 8)
    def _fill_zero(i):
        @pl.loop(0, D // 16)
        def _inner(k):
            zbuf[i, pl.ds(k * 16, 16)] = jnp.zeros((16,), jnp.float32)

    @pl.loop(0, RPT // 8)
    def _zero_acc(i):
        pltpu.sync_copy(zbuf, acc.at[pl.ds(s * RPT + i * 8, 8)])

    plsc.subcore_barrier()

    @pl.loop(0, AG)
    def _grp(g):
        pltpu.sync_copy(src_hbm.at[c, s, g], src_v)
        pltpu.sync_copy(dst_hbm.at[c, s, g], dst_v)
        for b in range(NBUF - 1):
            pltpu.async_copy(g_hbm.at[src_v.at[b]], rows[b], sems[b])

        @pl.loop(0, ACPG, step=NBUF)
        def _edges(j):
            for b in range(NBUF):
                pltpu.make_async_copy(
                    g_hbm.at[src_v.at[j + b]], rows[b], sems[b]).wait()

                @pl.when(j + b + NBUF - 1 < ACPG)
                def _issue():
                    nb = (b + NBUF - 1) % NBUF
                    pltpu.async_copy(
                        g_hbm.at[src_v.at[j + b + NBUF - 1]], rows[nb], sems[nb])

                pltpu.sync_copy(rows[b], acc.at[dst_v.at[j + b]], add=True)

    plsc.subcore_barrier()
    pltpu.sync_copy(acc.at[pl.ds(s * RPT, RPT)], out_hbm.at[c, pl.ds(s * RPT, RPT)])


def _tc_first_body(x_ref, w_ref, p0_ref, p1_ref, g_ref, dinv_ref):
    d = lax.rsqrt(p0_ref[...] + p1_ref[...] + 1.0)
    xw = jnp.dot(x_ref[...], w_ref[...], preferred_element_type=jnp.float32)
    g_ref[...] = xw * d
    dinv_ref[...] = d


def _tc_mid_body(p0_ref, p1_ref, g_ref, dinv_ref, b_ref, w_ref, gout_ref):
    d = dinv_ref[...]
    y = d * (p0_ref[...] + p1_ref[...] + g_ref[...]) + b_ref[...]
    h = jnp.maximum(y, 0.0)
    gout_ref[...] = jnp.dot(h, w_ref[...], preferred_element_type=jnp.float32) * d


def _tc_last_body(p0_ref, p1_ref, g_ref, dinv_ref, b_ref, out_ref):
    d = dinv_ref[...]
    out_ref[...] = d * (p0_ref[...] + p1_ref[...] + g_ref[...]) + b_ref[...]


def _rows(i):
    return (i, 0)


def _same(i):
    return (0, 0)


_b_rows = pl.BlockSpec((BM, D), _rows)
_b_col = pl.BlockSpec((BM, 1), _rows)
_b_w = pl.BlockSpec((D, D), _same)
_b_bias = pl.BlockSpec((1, D), _same)

_tc_first = pl.pallas_call(
    _tc_first_body,
    grid=(GRID,),
    in_specs=[_b_rows, _b_w, _b_col, _b_col],
    out_specs=[_b_rows, _b_col],
    out_shape=[
        jax.ShapeDtypeStruct((N, D), jnp.float32),
        jax.ShapeDtypeStruct((N, 1), jnp.float32),
    ],
)

_tc_mid = pl.pallas_call(
    _tc_mid_body,
    grid=(GRID,),
    in_specs=[_b_rows, _b_rows, _b_rows, _b_col, _b_bias, _b_w],
    out_specs=_b_rows,
    out_shape=jax.ShapeDtypeStruct((N, D), jnp.float32),
)

_tc_last = pl.pallas_call(
    _tc_last_body,
    grid=(GRID,),
    in_specs=[_b_rows, _b_rows, _b_rows, _b_col, _b_bias],
    out_specs=_b_rows,
    out_shape=jax.ShapeDtypeStruct((N, D), jnp.float32),
)


@jax.jit
def kernel(x, edge_index, W1, b1, W2, b2, W3, b3):
    ei = edge_index.astype(jnp.int32)
    src = ei[0].reshape(NC, NS, NCHUNK, K)
    dst = ei[1].reshape(NC, NS, NCHUNK, K)
    src_a = ei[0].reshape(NC, NS, AG, ACPG, AK)
    dst_a = ei[1].reshape(NC, NS, AG, ACPG, AK)

    deg = _deg_kernel(dst)
    p0 = deg[:N].reshape(N, 1)
    p1 = deg[NDEG:NDEG + N].reshape(N, 1)

    g1, dinv = _tc_first(x, W1, p0, p1)
    a1 = _agg_kernel(g1, src_a, dst_a)
    g2 = _tc_mid(a1[0], a1[1], g1, dinv, b1.reshape(1, D), W2)
    a2 = _agg_kernel(g2, src_a, dst_a)
    g3 = _tc_mid(a2[0], a2[1], g2, dinv, b2.reshape(1, D), W3)
    a3 = _agg_kernel(g3, src_a, dst_a)
    out = _tc_last(a3[0], a3[1], g3, dinv, b3.reshape(1, D))
    return out


# speedup vs baseline: 1.1503x; 1.0162x over previous
import functools

import jax
import jax.numpy as jnp
from jax import lax
from jax.experimental import pallas as pl
from jax.experimental.pallas import tpu as pltpu
from jax.experimental.pallas import tpu_sc as plsc

N = 10000
E = 320000
D = 128
NC, NS = 2, 16
EPT = E // (NC * NS)
K = 125
NCHUNK = EPT // K
CPG = 16
G = NCHUNK // CPG
AK = 50
ANCHUNK = EPT // AK
ACPG = 40
AG = ANCHUNK // ACPG
NACC = 10240
RPT = NACC // NS
NDEG = 10240
DPT = NDEG // NS
BM = 400
GRID = N // BM

_mesh = plsc.VectorSubcoreMesh(core_axis_name="c", subcore_axis_name="s")


@functools.partial(
    pl.kernel,
    mesh=_mesh,
    out_type=jax.ShapeDtypeStruct((NC * NDEG,), jnp.float32),
    scratch_types=[
        pltpu.VMEM((NCHUNK, K), jnp.int32),
        pltpu.VMEM((128,), jnp.float32),
        pltpu.VMEM((DPT,), jnp.float32),
        pltpu.VMEM_SHARED((NDEG,), jnp.float32),
    ],
)
def _deg_kernel(dst_hbm, out_hbm, dst_v, ones_v, zbuf, acc):
    c = lax.axis_index("c")
    s = lax.axis_index("s")
    pltpu.sync_copy(dst_hbm.at[c, s], dst_v)

    @pl.loop(0, 8)
    def _fill_ones(i):
        ones_v[pl.ds(i * 16, 16)] = jnp.ones((16,), jnp.float32)

    @pl.loop(0, DPT // 16)
    def _fill_zero(i):
        zbuf[pl.ds(i * 16, 16)] = jnp.zeros((16,), jnp.float32)

    pltpu.sync_copy(zbuf, acc.at[pl.ds(s * DPT, DPT)])
    plsc.subcore_barrier()

    @pl.loop(0, NCHUNK)
    def _scatter(j):
        pltpu.sync_copy(ones_v.at[pl.ds(0, K)], acc.at[dst_v.at[j]], add=True)

    plsc.subcore_barrier()
    pltpu.sync_copy(acc.at[pl.ds(s * DPT, DPT)],
                    out_hbm.at[pl.ds(c * NDEG + s * DPT, DPT)])


@functools.partial(
    pl.kernel,
    mesh=_mesh,
    out_type=jax.ShapeDtypeStruct((NC, NACC, D), jnp.float32),
    scratch_types=[
        pltpu.VMEM((ACPG, AK), jnp.int32),
        pltpu.VMEM((ACPG, AK), jnp.int32),
        pltpu.VMEM((AK, D), jnp.float32),
        pltpu.VMEM((AK, D), jnp.float32),
        pltpu.VMEM((AK, D), jnp.float32),
        pltpu.VMEM((AK, D), jnp.float32),
        pltpu.VMEM((RPT // 10, D), jnp.float32),
        pltpu.VMEM_SHARED((NACC, D), jnp.float32),
        pltpu.SemaphoreType.DMA,
        pltpu.SemaphoreType.DMA,
        pltpu.SemaphoreType.DMA,
        pltpu.SemaphoreType.DMA,
    ],
)
def _agg_kernel(g_hbm, src_hbm, dst_hbm, out_hbm,
                src_v, dst_v, rows0, rows1, rows2, rows3, zbuf, acc,
                sem0, sem1, sem2, sem3):
    c = lax.axis_index("c")
    s = lax.axis_index("s")

    @pl.loop(0, RPT // 10)
    def _fill_zero(i):
        @pl.loop(0, D // 16)
        def _inner(k):
            zbuf[i, pl.ds(k * 16, 16)] = jnp.zeros((16,), jnp.float32)

    @pl.loop(0, 10)
    def _zero_acc(i):
        pltpu.sync_copy(zbuf, acc.at[pl.ds(s * RPT + i * (RPT // 10), RPT // 10)])

    plsc.subcore_barrier()

    rows = (rows0, rows1, rows2, rows3)
    sems = (sem0, sem1, sem2, sem3)

    @pl.loop(0, AG)
    def _grp(g):
        pltpu.sync_copy(src_hbm.at[c, s, pl.ds(g * ACPG, ACPG)], src_v)
        pltpu.sync_copy(dst_hbm.at[c, s, pl.ds(g * ACPG, ACPG)], dst_v)
        for b in range(3):
            pltpu.async_copy(g_hbm.at[src_v.at[b]], rows[b], sems[b])

        @pl.loop(0, ACPG, step=4)
        def _edges(j):
            for b in range(4):
                pltpu.make_async_copy(
                    g_hbm.at[src_v.at[j + b]], rows[b], sems[b]).wait()

                @pl.when(j + b + 3 < ACPG)
                def _issue():
                    nb = (b + 3) % 4
                    pltpu.async_copy(
                        g_hbm.at[src_v.at[j + b + 3]], rows[nb], sems[nb])

                pltpu.sync_copy(rows[b], acc.at[dst_v.at[j + b]], add=True)

    plsc.subcore_barrier()
    pltpu.sync_copy(acc.at[pl.ds(s * RPT, RPT)], out_hbm.at[c, pl.ds(s * RPT, RPT)])


def _tc_first_body(x_ref, w_ref, p0_ref, p1_ref, g_ref, dinv_ref):
    d = lax.rsqrt(p0_ref[...] + p1_ref[...] + 1.0)
    xw = jnp.dot(x_ref[...], w_ref[...], preferred_element_type=jnp.float32)
    g_ref[...] = xw * d
    dinv_ref[...] = d


def _tc_mid_body(p0_ref, p1_ref, g_ref, dinv_ref, b_ref, w_ref, gout_ref):
    d = dinv_ref[...]
    y = d * (p0_ref[...] + p1_ref[...] + g_ref[...]) + b_ref[...]
    h = jnp.maximum(y, 0.0)
    gout_ref[...] = jnp.dot(h, w_ref[...], preferred_element_type=jnp.float32) * d


def _tc_last_body(p0_ref, p1_ref, g_ref, dinv_ref, b_ref, out_ref):
    d = dinv_ref[...]
    out_ref[...] = d * (p0_ref[...] + p1_ref[...] + g_ref[...]) + b_ref[...]


def _rows(i):
    return (i, 0)


def _same(i):
    return (0, 0)


_b_rows = pl.BlockSpec((BM, D), _rows)
_b_col = pl.BlockSpec((BM, 1), _rows)
_b_w = pl.BlockSpec((D, D), _same)
_b_bias = pl.BlockSpec((1, D), _same)

_tc_first = pl.pallas_call(
    _tc_first_body,
    grid=(GRID,),
    in_specs=[_b_rows, _b_w, _b_col, _b_col],
    out_specs=[_b_rows, _b_col],
    out_shape=[
        jax.ShapeDtypeStruct((N, D), jnp.float32),
        jax.ShapeDtypeStruct((N, 1), jnp.float32),
    ],
)

_tc_mid = pl.pallas_call(
    _tc_mid_body,
    grid=(GRID,),
    in_specs=[_b_rows, _b_rows, _b_rows, _b_col, _b_bias, _b_w],
    out_specs=_b_rows,
    out_shape=jax.ShapeDtypeStruct((N, D), jnp.float32),
)

_tc_last = pl.pallas_call(
    _tc_last_body,
    grid=(GRID,),
    in_specs=[_b_rows, _b_rows, _b_rows, _b_col, _b_bias],
    out_specs=_b_rows,
    out_shape=jax.ShapeDtypeStruct((N, D), jnp.float32),
)


@jax.jit
def kernel(x, edge_index, W1, b1, W2, b2, W3, b3):
    ei = edge_index.astype(jnp.int32)
    src = ei[0].reshape(NC, NS, NCHUNK, K)
    dst = ei[1].reshape(NC, NS, NCHUNK, K)
    src_a = ei[0].reshape(NC, NS, ANCHUNK, AK)
    dst_a = ei[1].reshape(NC, NS, ANCHUNK, AK)

    deg = _deg_kernel(dst)
    p0 = deg[:N].reshape(N, 1)
    p1 = deg[NDEG:NDEG + N].reshape(N, 1)

    g1, dinv = _tc_first(x, W1, p0, p1)
    a1 = _agg_kernel(g1, src_a, dst_a)
    g2 = _tc_mid(a1[0], a1[1], g1, dinv, b1.reshape(1, D), W2)
    a2 = _agg_kernel(g2, src_a, dst_a)
    g3 = _tc_mid(a2[0], a2[1], g2, dinv, b2.reshape(1, D), W3)
    a3 = _agg_kernel(g3, src_a, dst_a)
    out = _tc_last(a3[0], a3[1], g3, dinv, b3.reshape(1, D))
    return out


# speedup vs baseline: 1.2954x; 1.1261x over previous
import functools

import jax
import jax.numpy as jnp
from jax import lax
from jax.experimental import pallas as pl
from jax.experimental.pallas import tpu as pltpu
from jax.experimental.pallas import tpu_sc as plsc

N = 10000
E = 320000
D = 128
NC, NS = 2, 16
EPT = E // (NC * NS)
K = 125
NCHUNK = EPT // K
CPG = 16
G = NCHUNK // CPG
AK = 50
ANCHUNK = EPT // AK
ACPG = 40
AG = ANCHUNK // ACPG
NACC = 10240
RPT = NACC // NS
NDEG = 10240
DPT = NDEG // NS
BM = 1000
GRID = N // BM

_mesh = plsc.VectorSubcoreMesh(core_axis_name="c", subcore_axis_name="s")


@functools.partial(
    pl.kernel,
    mesh=_mesh,
    out_type=jax.ShapeDtypeStruct((NC * NDEG,), jnp.float32),
    scratch_types=[
        pltpu.VMEM((NCHUNK, K), jnp.int32),
        pltpu.VMEM((128,), jnp.float32),
        pltpu.VMEM((DPT,), jnp.float32),
        pltpu.VMEM_SHARED((NDEG,), jnp.float32),
    ],
)
def _deg_kernel(dst_hbm, out_hbm, dst_v, ones_v, zbuf, acc):
    c = lax.axis_index("c")
    s = lax.axis_index("s")
    pltpu.sync_copy(dst_hbm.at[c, s], dst_v)

    @pl.loop(0, 8)
    def _fill_ones(i):
        ones_v[pl.ds(i * 16, 16)] = jnp.ones((16,), jnp.float32)

    @pl.loop(0, DPT // 16)
    def _fill_zero(i):
        zbuf[pl.ds(i * 16, 16)] = jnp.zeros((16,), jnp.float32)

    pltpu.sync_copy(zbuf, acc.at[pl.ds(s * DPT, DPT)])
    plsc.subcore_barrier()

    @pl.loop(0, NCHUNK)
    def _scatter(j):
        pltpu.sync_copy(ones_v.at[pl.ds(0, K)], acc.at[dst_v.at[j]], add=True)

    plsc.subcore_barrier()
    pltpu.sync_copy(acc.at[pl.ds(s * DPT, DPT)],
                    out_hbm.at[pl.ds(c * NDEG + s * DPT, DPT)])


@functools.partial(
    pl.kernel,
    mesh=_mesh,
    out_type=jax.ShapeDtypeStruct((NC, NACC, D), jnp.float32),
    scratch_types=[
        pltpu.VMEM((ACPG, AK), jnp.int32),
        pltpu.VMEM((ACPG, AK), jnp.int32),
        pltpu.VMEM((AK, D), jnp.float32),
        pltpu.VMEM((AK, D), jnp.float32),
        pltpu.VMEM((AK, D), jnp.float32),
        pltpu.VMEM((AK, D), jnp.float32),
        pltpu.VMEM((RPT // 10, D), jnp.float32),
        pltpu.VMEM_SHARED((NACC, D), jnp.float32),
        pltpu.SemaphoreType.DMA,
        pltpu.SemaphoreType.DMA,
        pltpu.SemaphoreType.DMA,
        pltpu.SemaphoreType.DMA,
    ],
)
def _agg_kernel(g_hbm, src_hbm, dst_hbm, out_hbm,
                src_v, dst_v, rows0, rows1, rows2, rows3, zbuf, acc,
                sem0, sem1, sem2, sem3):
    c = lax.axis_index("c")
    s = lax.axis_index("s")

    @pl.loop(0, RPT // 10)
    def _fill_zero(i):
        @pl.loop(0, D // 16)
        def _inner(k):
            zbuf[i, pl.ds(k * 16, 16)] = jnp.zeros((16,), jnp.float32)

    @pl.loop(0, 10)
    def _zero_acc(i):
        pltpu.sync_copy(zbuf, acc.at[pl.ds(s * RPT + i * (RPT // 10), RPT // 10)])

    plsc.subcore_barrier()

    rows = (rows0, rows1, rows2, rows3)
    sems = (sem0, sem1, sem2, sem3)

    @pl.loop(0, AG)
    def _grp(g):
        pltpu.sync_copy(src_hbm.at[c, s, pl.ds(g * ACPG, ACPG)], src_v)
        pltpu.sync_copy(dst_hbm.at[c, s, pl.ds(g * ACPG, ACPG)], dst_v)
        for b in range(3):
            pltpu.async_copy(g_hbm.at[src_v.at[b]], rows[b], sems[b])

        @pl.loop(0, ACPG, step=4)
        def _edges(j):
            for b in range(4):
                pltpu.make_async_copy(
                    g_hbm.at[src_v.at[j + b]], rows[b], sems[b]).wait()

                @pl.when(j + b + 3 < ACPG)
                def _issue():
                    nb = (b + 3) % 4
                    pltpu.async_copy(
                        g_hbm.at[src_v.at[j + b + 3]], rows[nb], sems[nb])

                pltpu.sync_copy(rows[b], acc.at[dst_v.at[j + b]], add=True)

    plsc.subcore_barrier()
    pltpu.sync_copy(acc.at[pl.ds(s * RPT, RPT)], out_hbm.at[c, pl.ds(s * RPT, RPT)])


def _tc_first_body(x_ref, w_ref, p0_ref, p1_ref, g_ref, dinv_ref):
    d = lax.rsqrt(p0_ref[...] + p1_ref[...] + 1.0)
    xw = jnp.dot(x_ref[...], w_ref[...], preferred_element_type=jnp.float32)
    g_ref[...] = xw * d
    dinv_ref[...] = d


def _tc_mid_body(a_ref, g_ref, dinv_ref, b_ref, w_ref, gout_ref):
    d = dinv_ref[...]
    y = d * (a_ref[0] + a_ref[1] + g_ref[...]) + b_ref[...]
    h = jnp.maximum(y, 0.0)
    gout_ref[...] = jnp.dot(h, w_ref[...], preferred_element_type=jnp.float32) * d


def _tc_last_body(a_ref, g_ref, dinv_ref, b_ref, out_ref):
    d = dinv_ref[...]
    out_ref[...] = d * (a_ref[0] + a_ref[1] + g_ref[...]) + b_ref[...]


def _rows(i):
    return (i, 0)


def _same(i):
    return (0, 0)


_b_rows = pl.BlockSpec((BM, D), _rows)
_b_parts = pl.BlockSpec((NC, BM, D), lambda i: (0, i, 0))
_b_col = pl.BlockSpec((BM, 1), _rows)
_b_w = pl.BlockSpec((D, D), _same)
_b_bias = pl.BlockSpec((1, D), _same)

_tc_first = pl.pallas_call(
    _tc_first_body,
    grid=(GRID,),
    in_specs=[_b_rows, _b_w, _b_col, _b_col],
    out_specs=[_b_rows, _b_col],
    out_shape=[
        jax.ShapeDtypeStruct((N, D), jnp.float32),
        jax.ShapeDtypeStruct((N, 1), jnp.float32),
    ],
)

_tc_mid = pl.pallas_call(
    _tc_mid_body,
    grid=(GRID,),
    in_specs=[_b_parts, _b_rows, _b_col, _b_bias, _b_w],
    out_specs=_b_rows,
    out_shape=jax.ShapeDtypeStruct((N, D), jnp.float32),
)

_tc_last = pl.pallas_call(
    _tc_last_body,
    grid=(GRID,),
    in_specs=[_b_parts, _b_rows, _b_col, _b_bias],
    out_specs=_b_rows,
    out_shape=jax.ShapeDtypeStruct((N, D), jnp.float32),
)


@jax.jit
def kernel(x, edge_index, W1, b1, W2, b2, W3, b3):
    ei = edge_index.astype(jnp.int32)
    src = ei[0].reshape(NC, NS, NCHUNK, K)
    dst = ei[1].reshape(NC, NS, NCHUNK, K)
    src_a = ei[0].reshape(NC, NS, ANCHUNK, AK)
    dst_a = ei[1].reshape(NC, NS, ANCHUNK, AK)

    deg = _deg_kernel(dst)
    p0 = deg[:N].reshape(N, 1)
    p1 = deg[NDEG:NDEG + N].reshape(N, 1)

    g1, dinv = _tc_first(x, W1, p0, p1)
    a1 = _agg_kernel(g1, src_a, dst_a)
    g2 = _tc_mid(a1, g1, dinv, b1.reshape(1, D), W2)
    a2 = _agg_kernel(g2, src_a, dst_a)
    g3 = _tc_mid(a2, g2, dinv, b2.reshape(1, D), W3)
    a3 = _agg_kernel(g3, src_a, dst_a)
    out = _tc_last(a3, g3, dinv, b3.reshape(1, D))
    return out


# speedup vs baseline: 1.3206x; 1.0194x over previous
import functools

import jax
import jax.numpy as jnp
from jax import lax
from jax.experimental import pallas as pl
from jax.experimental.pallas import tpu as pltpu
from jax.experimental.pallas import tpu_sc as plsc

N = 10000
E = 320000
D = 128
NC, NS = 2, 16
EPT = E // (NC * NS)
K = 125
NCHUNK = EPT // K
CPG = 16
G = NCHUNK // CPG
AK = 50
ANCHUNK = EPT // AK
ACPG = 40
AG = ANCHUNK // ACPG
NACC = 10240
RPT = NACC // NS
NDEG = 10240
DPT = NDEG // NS
BM = 2000
GRID = N // BM

_mesh = plsc.VectorSubcoreMesh(core_axis_name="c", subcore_axis_name="s")


@functools.partial(
    pl.kernel,
    mesh=_mesh,
    out_type=jax.ShapeDtypeStruct((NC * NDEG,), jnp.float32),
    scratch_types=[
        pltpu.VMEM((NCHUNK, K), jnp.int32),
        pltpu.VMEM((128,), jnp.float32),
        pltpu.VMEM((DPT,), jnp.float32),
        pltpu.VMEM_SHARED((NDEG,), jnp.float32),
    ],
)
def _deg_kernel(dst_hbm, out_hbm, dst_v, ones_v, zbuf, acc):
    c = lax.axis_index("c")
    s = lax.axis_index("s")
    pltpu.sync_copy(dst_hbm.at[c, s], dst_v)

    @pl.loop(0, 8)
    def _fill_ones(i):
        ones_v[pl.ds(i * 16, 16)] = jnp.ones((16,), jnp.float32)

    @pl.loop(0, DPT // 16)
    def _fill_zero(i):
        zbuf[pl.ds(i * 16, 16)] = jnp.zeros((16,), jnp.float32)

    pltpu.sync_copy(zbuf, acc.at[pl.ds(s * DPT, DPT)])
    plsc.subcore_barrier()

    @pl.loop(0, NCHUNK)
    def _scatter(j):
        pltpu.sync_copy(ones_v.at[pl.ds(0, K)], acc.at[dst_v.at[j]], add=True)

    plsc.subcore_barrier()
    pltpu.sync_copy(acc.at[pl.ds(s * DPT, DPT)],
                    out_hbm.at[pl.ds(c * NDEG + s * DPT, DPT)])


@functools.partial(
    pl.kernel,
    mesh=_mesh,
    out_type=jax.ShapeDtypeStruct((NC, NACC, D), jnp.float32),
    scratch_types=[
        pltpu.VMEM((ACPG, AK), jnp.int32),
        pltpu.VMEM((ACPG, AK), jnp.int32),
        pltpu.VMEM((AK, D), jnp.float32),
        pltpu.VMEM((AK, D), jnp.float32),
        pltpu.VMEM((AK, D), jnp.float32),
        pltpu.VMEM((AK, D), jnp.float32),
        pltpu.VMEM((RPT // 10, D), jnp.float32),
        pltpu.VMEM_SHARED((NACC, D), jnp.float32),
        pltpu.SemaphoreType.DMA,
        pltpu.SemaphoreType.DMA,
        pltpu.SemaphoreType.DMA,
        pltpu.SemaphoreType.DMA,
    ],
)
def _agg_kernel(g_hbm, src_hbm, dst_hbm, out_hbm,
                src_v, dst_v, rows0, rows1, rows2, rows3, zbuf, acc,
                sem0, sem1, sem2, sem3):
    c = lax.axis_index("c")
    s = lax.axis_index("s")

    @pl.loop(0, RPT // 10)
    def _fill_zero(i):
        @pl.loop(0, D // 16)
        def _inner(k):
            zbuf[i, pl.ds(k * 16, 16)] = jnp.zeros((16,), jnp.float32)

    @pl.loop(0, 10)
    def _zero_acc(i):
        pltpu.sync_copy(zbuf, acc.at[pl.ds(s * RPT + i * (RPT // 10), RPT // 10)])

    plsc.subcore_barrier()

    rows = (rows0, rows1, rows2, rows3)
    sems = (sem0, sem1, sem2, sem3)

    @pl.loop(0, AG)
    def _grp(g):
        pltpu.sync_copy(src_hbm.at[c, s, pl.ds(g * ACPG, ACPG)], src_v)
        pltpu.sync_copy(dst_hbm.at[c, s, pl.ds(g * ACPG, ACPG)], dst_v)
        for b in range(3):
            pltpu.async_copy(g_hbm.at[src_v.at[b]], rows[b], sems[b])

        @pl.loop(0, ACPG, step=4)
        def _edges(j):
            for b in range(4):
                pltpu.make_async_copy(
                    g_hbm.at[src_v.at[j + b]], rows[b], sems[b]).wait()

                @pl.when(j + b + 3 < ACPG)
                def _issue():
                    nb = (b + 3) % 4
                    pltpu.async_copy(
                        g_hbm.at[src_v.at[j + b + 3]], rows[nb], sems[nb])

                pltpu.sync_copy(rows[b], acc.at[dst_v.at[j + b]], add=True)

    plsc.subcore_barrier()
    pltpu.sync_copy(acc.at[pl.ds(s * RPT, RPT)], out_hbm.at[c, pl.ds(s * RPT, RPT)])


def _tc_first_body(x_ref, w_ref, p0_ref, p1_ref, g_ref, dinv_ref):
    d = lax.rsqrt(p0_ref[...] + p1_ref[...] + 1.0)
    xw = jnp.dot(x_ref[...], w_ref[...], preferred_element_type=jnp.float32)
    g_ref[...] = xw * d
    dinv_ref[...] = d


def _tc_mid_body(a_ref, g_ref, dinv_ref, b_ref, w_ref, gout_ref):
    d = dinv_ref[...]
    y = d * (a_ref[0] + a_ref[1] + g_ref[...]) + b_ref[...]
    h = jnp.maximum(y, 0.0)
    gout_ref[...] = jnp.dot(h, w_ref[...], preferred_element_type=jnp.float32) * d


def _tc_last_body(a_ref, g_ref, dinv_ref, b_ref, out_ref):
    d = dinv_ref[...]
    out_ref[...] = d * (a_ref[0] + a_ref[1] + g_ref[...]) + b_ref[...]


def _rows(i):
    return (i, 0)


def _same(i):
    return (0, 0)


_b_rows = pl.BlockSpec((BM, D), _rows)
_b_parts = pl.BlockSpec((NC, BM, D), lambda i: (0, i, 0))
_b_col = pl.BlockSpec((BM, 1), _rows)
_b_w = pl.BlockSpec((D, D), _same)
_b_bias = pl.BlockSpec((1, D), _same)

_tc_first = pl.pallas_call(
    _tc_first_body,
    grid=(GRID,),
    in_specs=[_b_rows, _b_w, _b_col, _b_col],
    out_specs=[_b_rows, _b_col],
    out_shape=[
        jax.ShapeDtypeStruct((N, D), jnp.float32),
        jax.ShapeDtypeStruct((N, 1), jnp.float32),
    ],
)

_tc_mid = pl.pallas_call(
    _tc_mid_body,
    grid=(GRID,),
    in_specs=[_b_parts, _b_rows, _b_col, _b_bias, _b_w],
    out_specs=_b_rows,
    out_shape=jax.ShapeDtypeStruct((N, D), jnp.float32),
)

_tc_last = pl.pallas_call(
    _tc_last_body,
    grid=(GRID,),
    in_specs=[_b_parts, _b_rows, _b_col, _b_bias],
    out_specs=_b_rows,
    out_shape=jax.ShapeDtypeStruct((N, D), jnp.float32),
)


@jax.jit
def kernel(x, edge_index, W1, b1, W2, b2, W3, b3):
    ei = edge_index.astype(jnp.int32)
    src = ei[0].reshape(NC, NS, NCHUNK, K)
    dst = ei[1].reshape(NC, NS, NCHUNK, K)
    src_a = ei[0].reshape(NC, NS, ANCHUNK, AK)
    dst_a = ei[1].reshape(NC, NS, ANCHUNK, AK)

    deg = _deg_kernel(dst)
    p0 = deg[:N].reshape(N, 1)
    p1 = deg[NDEG:NDEG + N].reshape(N, 1)

    g1, dinv = _tc_first(x, W1, p0, p1)
    a1 = _agg_kernel(g1, src_a, dst_a)
    g2 = _tc_mid(a1, g1, dinv, b1.reshape(1, D), W2)
    a2 = _agg_kernel(g2, src_a, dst_a)
    g3 = _tc_mid(a2, g2, dinv, b2.reshape(1, D), W3)
    a3 = _agg_kernel(g3, src_a, dst_a)
    out = _tc_last(a3, g3, dinv, b3.reshape(1, D))
    return out


# speedup vs baseline: 1.4239x; 1.0782x over previous
import functools

import jax
import jax.numpy as jnp
from jax import lax
from jax.experimental import pallas as pl
from jax.experimental.pallas import tpu as pltpu
from jax.experimental.pallas import tpu_sc as plsc

N = 10000
E = 320000
D = 128
NC, NS = 2, 16
EPT = E // (NC * NS)
K = 125
NCHUNK = EPT // K
CPG = 16
G = NCHUNK // CPG
AK = 50
ANCHUNK = EPT // AK
ACPG = 40
AG = ANCHUNK // ACPG
NBUF = 4
NACC = 10112
RPT = NACC // NS
NDEG = 10240
DPT = NDEG // NS
BM = 2000
GRID = N // BM

_mesh = plsc.VectorSubcoreMesh(core_axis_name="c", subcore_axis_name="s")


@functools.partial(
    pl.kernel,
    mesh=_mesh,
    out_type=jax.ShapeDtypeStruct((NC * NDEG,), jnp.float32),
    scratch_types=[
        pltpu.VMEM((NCHUNK, K), jnp.int32),
        pltpu.VMEM((128,), jnp.float32),
        pltpu.VMEM((DPT,), jnp.float32),
        pltpu.VMEM_SHARED((NDEG,), jnp.float32),
    ],
)
def _deg_kernel(dst_hbm, out_hbm, dst_v, ones_v, zbuf, acc):
    c = lax.axis_index("c")
    s = lax.axis_index("s")
    pltpu.sync_copy(dst_hbm.at[c, s], dst_v)

    @pl.loop(0, 8)
    def _fill_ones(i):
        ones_v[pl.ds(i * 16, 16)] = jnp.ones((16,), jnp.float32)

    @pl.loop(0, DPT // 16)
    def _fill_zero(i):
        zbuf[pl.ds(i * 16, 16)] = jnp.zeros((16,), jnp.float32)

    pltpu.sync_copy(zbuf, acc.at[pl.ds(s * DPT, DPT)])
    plsc.subcore_barrier()

    @pl.loop(0, NCHUNK)
    def _scatter(j):
        pltpu.sync_copy(ones_v.at[pl.ds(0, K)], acc.at[dst_v.at[j]], add=True)

    plsc.subcore_barrier()
    pltpu.sync_copy(acc.at[pl.ds(s * DPT, DPT)],
                    out_hbm.at[pl.ds(c * NDEG + s * DPT, DPT)])


@functools.partial(
    pl.kernel,
    mesh=_mesh,
    out_type=jax.ShapeDtypeStruct((NC, NACC, D), jnp.float32),
    scratch_types=(
        [pltpu.VMEM((ACPG, AK), jnp.int32)] * 4
        + [pltpu.VMEM((AK, D), jnp.float32)] * NBUF
        + [pltpu.VMEM_SHARED((NACC, D), jnp.float32)]
        + [pltpu.SemaphoreType.DMA] * (NBUF + 1)
    ),
)
def _agg_kernel(g_hbm, src_hbm, dst_hbm, out_hbm, *refs):
    src0, dst0, src1, dst1 = refs[0:4]
    rows = refs[4:4 + NBUF]
    acc = refs[4 + NBUF]
    sems = refs[5 + NBUF:5 + 2 * NBUF]
    isem = refs[5 + 2 * NBUF]
    c = lax.axis_index("c")
    s = lax.axis_index("s")

    pltpu.sync_copy(src_hbm.at[c, s, 0], src0)
    pltpu.sync_copy(dst_hbm.at[c, s, 0], dst0)
    for b in range(NBUF - 1):
        pltpu.async_copy(g_hbm.at[src0.at[b]], rows[b], sems[b])

    zbuf = rows[NBUF - 1]

    @pl.loop(0, AK)
    def _fill_zero(i):
        @pl.loop(0, D // 16)
        def _inner(k):
            zbuf[i, pl.ds(k * 16, 16)] = jnp.zeros((16,), jnp.float32)

    @pl.loop(0, RPT // AK)
    def _zero_acc(i):
        pltpu.sync_copy(zbuf, acc.at[pl.ds(s * RPT + i * AK, AK)])

    pltpu.sync_copy(zbuf.at[pl.ds(0, RPT - (RPT // AK) * AK)],
                    acc.at[pl.ds(s * RPT + (RPT // AK) * AK,
                                 RPT - (RPT // AK) * AK)])
    plsc.subcore_barrier()

    def _group(g, cs, cd, nsrc, ndst):
        @pl.when(g + 1 < AG)
        def _prefetch():
            pltpu.async_copy(src_hbm.at[c, s, g + 1], nsrc, isem)
            pltpu.async_copy(dst_hbm.at[c, s, g + 1], ndst, isem)

        @pl.loop(0, ACPG, step=NBUF)
        def _inner(j):
            @pl.when(jnp.logical_and(j == ACPG - NBUF, g + 1 < AG))
            def _wait_idx():
                pltpu.make_async_copy(src_hbm.at[c, s, g + 1], nsrc, isem).wait()
                pltpu.make_async_copy(dst_hbm.at[c, s, g + 1], ndst, isem).wait()

            for b in range(NBUF):
                pltpu.make_async_copy(
                    g_hbm.at[cs.at[j + b]], rows[b], sems[b]).wait()
                nb = (b + NBUF - 1) % NBUF

                @pl.when(j + b + NBUF - 1 < ACPG)
                def _issue_in():
                    pltpu.async_copy(
                        g_hbm.at[cs.at[j + b + NBUF - 1]], rows[nb], sems[nb])

                @pl.when(jnp.logical_and(j + b + NBUF - 1 >= ACPG,
                                         g + 1 < AG))
                def _issue_cross():
                    pltpu.async_copy(
                        g_hbm.at[nsrc.at[j + b + NBUF - 1 - ACPG]],
                        rows[nb], sems[nb])

                pltpu.sync_copy(rows[b], acc.at[cd.at[j + b]], add=True)

    @pl.loop(0, AG, step=2)
    def _gpair(g):
        _group(g, src0, dst0, src1, dst1)

        @pl.when(g + 1 < AG)
        def _second():
            _group(g + 1, src1, dst1, src0, dst0)

    plsc.subcore_barrier()
    pltpu.sync_copy(acc.at[pl.ds(s * RPT, RPT)], out_hbm.at[c, pl.ds(s * RPT, RPT)])


def _tc_first_body(x_ref, w_ref, p0_ref, p1_ref, g_ref, dinv_ref):
    d = lax.rsqrt(p0_ref[...] + p1_ref[...] + 1.0)
    xw = jnp.dot(x_ref[...], w_ref[...], preferred_element_type=jnp.float32)
    g_ref[...] = xw * d
    dinv_ref[...] = d


def _tc_mid_body(a_ref, g_ref, dinv_ref, b_ref, w_ref, gout_ref):
    d = dinv_ref[...]
    y = d * (a_ref[0] + a_ref[1] + g_ref[...]) + b_ref[...]
    h = jnp.maximum(y, 0.0)
    gout_ref[...] = jnp.dot(h, w_ref[...], preferred_element_type=jnp.float32) * d


def _tc_last_body(a_ref, g_ref, dinv_ref, b_ref, out_ref):
    d = dinv_ref[...]
    out_ref[...] = d * (a_ref[0] + a_ref[1] + g_ref[...]) + b_ref[...]


def _rows(i):
    return (i, 0)


def _same(i):
    return (0, 0)


_b_rows = pl.BlockSpec((BM, D), _rows)
_b_parts = pl.BlockSpec((NC, BM, D), lambda i: (0, i, 0))
_b_col = pl.BlockSpec((BM, 1), _rows)
_b_w = pl.BlockSpec((D, D), _same)
_b_bias = pl.BlockSpec((1, D), _same)

_tc_first = pl.pallas_call(
    _tc_first_body,
    grid=(GRID,),
    in_specs=[_b_rows, _b_w, _b_col, _b_col],
    out_specs=[_b_rows, _b_col],
    out_shape=[
        jax.ShapeDtypeStruct((N, D), jnp.float32),
        jax.ShapeDtypeStruct((N, 1), jnp.float32),
    ],
)

_tc_mid = pl.pallas_call(
    _tc_mid_body,
    grid=(GRID,),
    in_specs=[_b_parts, _b_rows, _b_col, _b_bias, _b_w],
    out_specs=_b_rows,
    out_shape=jax.ShapeDtypeStruct((N, D), jnp.float32),
)

_tc_last = pl.pallas_call(
    _tc_last_body,
    grid=(GRID,),
    in_specs=[_b_parts, _b_rows, _b_col, _b_bias],
    out_specs=_b_rows,
    out_shape=jax.ShapeDtypeStruct((N, D), jnp.float32),
)


@jax.jit
def kernel(x, edge_index, W1, b1, W2, b2, W3, b3):
    ei = edge_index.astype(jnp.int32)
    src = ei[0].reshape(NC, NS, NCHUNK, K)
    dst = ei[1].reshape(NC, NS, NCHUNK, K)
    src_a = ei[0].reshape(NC, NS, AG, ACPG, AK)
    dst_a = ei[1].reshape(NC, NS, AG, ACPG, AK)

    deg = _deg_kernel(dst)
    p0 = deg[:N].reshape(N, 1)
    p1 = deg[NDEG:NDEG + N].reshape(N, 1)

    g1, dinv = _tc_first(x, W1, p0, p1)
    a1 = _agg_kernel(g1, src_a, dst_a)
    g2 = _tc_mid(a1, g1, dinv, b1.reshape(1, D), W2)
    a2 = _agg_kernel(g2, src_a, dst_a)
    g3 = _tc_mid(a2, g2, dinv, b2.reshape(1, D), W3)
    a3 = _agg_kernel(g3, src_a, dst_a)
    out = _tc_last(a3, g3, dinv, b3.reshape(1, D))
    return out


# speedup vs baseline: 1.4342x; 1.0072x over previous
import functools

import jax
import jax.numpy as jnp
from jax import lax
from jax.experimental import pallas as pl
from jax.experimental.pallas import tpu as pltpu
from jax.experimental.pallas import tpu_sc as plsc

N = 10000
E = 320000
D = 128
NC, NS = 2, 16
EPT = E // (NC * NS)
K = 125
NCHUNK = EPT // K
CPG = 16
G = NCHUNK // CPG
AK = 50
ANCHUNK = EPT // AK
ACPG = 40
AG = ANCHUNK // ACPG
NBUF = 4
NACC = 10112
RPT = NACC // NS
NDEG = 10240
DPT = NDEG // NS
BM = 5000
GRID = N // BM

_mesh = plsc.VectorSubcoreMesh(core_axis_name="c", subcore_axis_name="s")


@functools.partial(
    pl.kernel,
    mesh=_mesh,
    out_type=jax.ShapeDtypeStruct((NC * NDEG,), jnp.float32),
    scratch_types=[
        pltpu.VMEM((NCHUNK, K), jnp.int32),
        pltpu.VMEM((128,), jnp.float32),
        pltpu.VMEM((DPT,), jnp.float32),
        pltpu.VMEM_SHARED((NDEG,), jnp.float32),
    ],
)
def _deg_kernel(dst_hbm, out_hbm, dst_v, ones_v, zbuf, acc):
    c = lax.axis_index("c")
    s = lax.axis_index("s")
    pltpu.sync_copy(dst_hbm.at[c, s], dst_v)

    @pl.loop(0, 8)
    def _fill_ones(i):
        ones_v[pl.ds(i * 16, 16)] = jnp.ones((16,), jnp.float32)

    @pl.loop(0, DPT // 16)
    def _fill_zero(i):
        zbuf[pl.ds(i * 16, 16)] = jnp.zeros((16,), jnp.float32)

    pltpu.sync_copy(zbuf, acc.at[pl.ds(s * DPT, DPT)])
    plsc.subcore_barrier()

    @pl.loop(0, NCHUNK)
    def _scatter(j):
        pltpu.sync_copy(ones_v.at[pl.ds(0, K)], acc.at[dst_v.at[j]], add=True)

    plsc.subcore_barrier()
    pltpu.sync_copy(acc.at[pl.ds(s * DPT, DPT)],
                    out_hbm.at[pl.ds(c * NDEG + s * DPT, DPT)])


@functools.partial(
    pl.kernel,
    mesh=_mesh,
    out_type=jax.ShapeDtypeStruct((NC, NACC, D), jnp.float32),
    scratch_types=(
        [pltpu.VMEM((ACPG, AK), jnp.int32)] * 4
        + [pltpu.VMEM((AK, D), jnp.float32)] * NBUF
        + [pltpu.VMEM_SHARED((NACC, D), jnp.float32)]
        + [pltpu.SemaphoreType.DMA] * (NBUF + 1)
    ),
)
def _agg_kernel(g_hbm, src_hbm, dst_hbm, out_hbm, *refs):
    src0, dst0, src1, dst1 = refs[0:4]
    rows = refs[4:4 + NBUF]
    acc = refs[4 + NBUF]
    sems = refs[5 + NBUF:5 + 2 * NBUF]
    isem = refs[5 + 2 * NBUF]
    c = lax.axis_index("c")
    s = lax.axis_index("s")

    pltpu.sync_copy(src_hbm.at[c, s, 0], src0)
    pltpu.sync_copy(dst_hbm.at[c, s, 0], dst0)
    for b in range(NBUF - 1):
        pltpu.async_copy(g_hbm.at[src0.at[b]], rows[b], sems[b])

    zbuf = rows[NBUF - 1]

    @pl.loop(0, AK)
    def _fill_zero(i):
        @pl.loop(0, D // 16)
        def _inner(k):
            zbuf[i, pl.ds(k * 16, 16)] = jnp.zeros((16,), jnp.float32)

    @pl.loop(0, RPT // AK)
    def _zero_acc(i):
        pltpu.sync_copy(zbuf, acc.at[pl.ds(s * RPT + i * AK, AK)])

    pltpu.sync_copy(zbuf.at[pl.ds(0, RPT - (RPT // AK) * AK)],
                    acc.at[pl.ds(s * RPT + (RPT // AK) * AK,
                                 RPT - (RPT // AK) * AK)])
    plsc.subcore_barrier()

    def _group(g, cs, cd, nsrc, ndst):
        @pl.when(g + 1 < AG)
        def _prefetch():
            pltpu.async_copy(src_hbm.at[c, s, g + 1], nsrc, isem)
            pltpu.async_copy(dst_hbm.at[c, s, g + 1], ndst, isem)

        @pl.loop(0, ACPG, step=NBUF)
        def _inner(j):
            @pl.when(jnp.logical_and(j == ACPG - NBUF, g + 1 < AG))
            def _wait_idx():
                pltpu.make_async_copy(src_hbm.at[c, s, g + 1], nsrc, isem).wait()
                pltpu.make_async_copy(dst_hbm.at[c, s, g + 1], ndst, isem).wait()

            for b in range(NBUF):
                pltpu.make_async_copy(
                    g_hbm.at[cs.at[j + b]], rows[b], sems[b]).wait()
                nb = (b + NBUF - 1) % NBUF

                @pl.when(j + b + NBUF - 1 < ACPG)
                def _issue_in():
                    pltpu.async_copy(
                        g_hbm.at[cs.at[j + b + NBUF - 1]], rows[nb], sems[nb])

                @pl.when(jnp.logical_and(j + b + NBUF - 1 >= ACPG,
                                         g + 1 < AG))
                def _issue_cross():
                    pltpu.async_copy(
                        g_hbm.at[nsrc.at[j + b + NBUF - 1 - ACPG]],
                        rows[nb], sems[nb])

                pltpu.sync_copy(rows[b], acc.at[cd.at[j + b]], add=True)

    @pl.loop(0, AG, step=2)
    def _gpair(g):
        _group(g, src0, dst0, src1, dst1)

        @pl.when(g + 1 < AG)
        def _second():
            _group(g + 1, src1, dst1, src0, dst0)

    plsc.subcore_barrier()
    pltpu.sync_copy(acc.at[pl.ds(s * RPT, RPT)], out_hbm.at[c, pl.ds(s * RPT, RPT)])


def _tc_first_body(x_ref, w_ref, p0_ref, p1_ref, g_ref, dinv_ref):
    d = lax.rsqrt(p0_ref[...] + p1_ref[...] + 1.0)
    xw = jnp.dot(x_ref[...], w_ref[...], preferred_element_type=jnp.float32)
    g_ref[...] = xw * d
    dinv_ref[...] = d


def _tc_mid_body(a_ref, g_ref, dinv_ref, b_ref, w_ref, gout_ref):
    d = dinv_ref[...]
    y = d * (a_ref[0] + a_ref[1] + g_ref[...]) + b_ref[...]
    h = jnp.maximum(y, 0.0)
    gout_ref[...] = jnp.dot(h, w_ref[...], preferred_element_type=jnp.float32) * d


def _tc_last_body(a_ref, g_ref, dinv_ref, b_ref, out_ref):
    d = dinv_ref[...]
    out_ref[...] = d * (a_ref[0] + a_ref[1] + g_ref[...]) + b_ref[...]


def _rows(i):
    return (i, 0)


def _same(i):
    return (0, 0)


_b_rows = pl.BlockSpec((BM, D), _rows)
_b_parts = pl.BlockSpec((NC, BM, D), lambda i: (0, i, 0))
_b_col = pl.BlockSpec((BM, 1), _rows)
_b_w = pl.BlockSpec((D, D), _same)
_b_bias = pl.BlockSpec((1, D), _same)

_tc_first = pl.pallas_call(
    _tc_first_body,
    grid=(GRID,),
    in_specs=[_b_rows, _b_w, _b_col, _b_col],
    out_specs=[_b_rows, _b_col],
    out_shape=[
        jax.ShapeDtypeStruct((N, D), jnp.float32),
        jax.ShapeDtypeStruct((N, 1), jnp.float32),
    ],
)

_tc_mid = pl.pallas_call(
    _tc_mid_body,
    grid=(GRID,),
    in_specs=[_b_parts, _b_rows, _b_col, _b_bias, _b_w],
    out_specs=_b_rows,
    out_shape=jax.ShapeDtypeStruct((N, D), jnp.float32),
)

_tc_last = pl.pallas_call(
    _tc_last_body,
    grid=(GRID,),
    in_specs=[_b_parts, _b_rows, _b_col, _b_bias],
    out_specs=_b_rows,
    out_shape=jax.ShapeDtypeStruct((N, D), jnp.float32),
)


@jax.jit
def kernel(x, edge_index, W1, b1, W2, b2, W3, b3):
    ei = edge_index.astype(jnp.int32)
    src = ei[0].reshape(NC, NS, NCHUNK, K)
    dst = ei[1].reshape(NC, NS, NCHUNK, K)
    src_a = ei[0].reshape(NC, NS, AG, ACPG, AK)
    dst_a = ei[1].reshape(NC, NS, AG, ACPG, AK)

    deg = _deg_kernel(dst)
    p0 = deg[:N].reshape(N, 1)
    p1 = deg[NDEG:NDEG + N].reshape(N, 1)

    g1, dinv = _tc_first(x, W1, p0, p1)
    a1 = _agg_kernel(g1, src_a, dst_a)
    g2 = _tc_mid(a1, g1, dinv, b1.reshape(1, D), W2)
    a2 = _agg_kernel(g2, src_a, dst_a)
    g3 = _tc_mid(a2, g2, dinv, b2.reshape(1, D), W3)
    a3 = _agg_kernel(g3, src_a, dst_a)
    out = _tc_last(a3, g3, dinv, b3.reshape(1, D))
    return out
